# Initial kernel scaffold; baseline (speedup 1.0000x reference)
import jax
import jax.numpy as jnp
from jax.experimental import pallas as pl


def kernel(src_walks, tgt_walks, src_lens, tgt_lens, own_emb, cross_emb):
    raise NotImplementedError("write your pallas kernel here")


# speedup vs baseline: 5.2859x; 5.2859x over previous
import functools

import jax
import jax.numpy as jnp
from jax import lax
from jax.experimental import pallas as pl


def _body(L, M, W, R, flat_ref, col_ref, pos_ref, emb_ref, src_out_ref, tgt_out_ref):
    SENT = L
    ZROW = L + 1
    flat_blk = flat_ref[...]
    pos_blk = pos_ref[...]
    col_blk = col_ref[...]
    emb = emb_ref[...]

    ridx = lax.broadcasted_iota(jnp.int32, (2 * W, 1), 0)
    is_src = ridx < W
    jcol = lax.broadcasted_iota(jnp.int32, (2 * W, 64), 1)

    for r in range(R):
        fl_row = flat_blk[r:r + 1, :]
        pos_row = pos_blk[r:r + 1, :]
        fl_col = col_blk[r]
        eq = fl_col == fl_row
        dflt = jnp.where(fl_col < 0, ZROW, SENT)
        vals = jnp.where(eq, pos_row, dflt)
        minA = jnp.min(vals[:, :W], axis=1, keepdims=True)
        minB = jnp.min(vals[:, W:], axis=1, keepdims=True)
        own = jnp.where(is_src, minA, minB)
        cross = jnp.where(is_src, minB, minA)
        oh = ((jcol == own) | (jcol == cross + 32)).astype(jnp.float32)
        out = jnp.dot(oh, emb, preferred_element_type=jnp.float32)
        src_out_ref[r] = out[0:M, :]
        tgt_out_ref[r] = out[W:W + M, :]


def kernel(src_walks, tgt_walks, src_lens, tgt_lens, own_emb, cross_emb):
    B, K, L = src_walks.shape
    M = K * L
    POS_DIM = own_emb.shape[1] + cross_emb.shape[1]
    SENT = L
    ZROW = L + 1
    PAD = (-M) % 128 or 64
    W = M + PAD
    R = 8

    src_walks = src_walks.astype(jnp.int32)
    tgt_walks = tgt_walks.astype(jnp.int32)
    pos_grid = jnp.arange(L, dtype=jnp.int32).reshape(1, 1, L)
    src_valid = (pos_grid < src_lens[..., None]) & (src_walks != 0)
    tgt_valid = (pos_grid < tgt_lens[..., None]) & (tgt_walks != 0)

    pos_flat = jnp.broadcast_to(
        jnp.tile(jnp.arange(L, dtype=jnp.int32), K).reshape(1, M), (B, M))

    def side(flat, valid):
        fl = jnp.where(valid.reshape(B, M), flat.reshape(B, M), -1)
        fl = jnp.concatenate(
            [fl, jnp.full((B, PAD), -1, jnp.int32)], axis=1)
        ps = jnp.where(valid.reshape(B, M), pos_flat, ZROW)
        ps = jnp.concatenate(
            [ps, jnp.full((B, PAD), ZROW, jnp.int32)], axis=1)
        return fl, ps

    sfl, sps = side(src_walks, src_valid)
    tfl, tps = side(tgt_walks, tgt_valid)
    cat_flat = jnp.concatenate([sfl, tfl], axis=1)
    cat_pos = jnp.concatenate([sps, tps], axis=1)
    cat_col = cat_flat[..., None]

    HALF = own_emb.shape[1]
    emb_mat = (jnp.zeros((64, POS_DIM), jnp.float32)
               .at[0:L + 1, 0:HALF].set(own_emb[:L + 1])
               .at[32:32 + L + 1, HALF:POS_DIM].set(cross_emb[:L + 1]))

    grid = (B // R,)
    out_shape = [
        jax.ShapeDtypeStruct((B, M, POS_DIM), jnp.float32),
        jax.ShapeDtypeStruct((B, M, POS_DIM), jnp.float32),
    ]
    src_pos, tgt_pos = pl.pallas_call(
        functools.partial(_body, L, M, W, R),
        grid=grid,
        in_specs=[
            pl.BlockSpec((R, 2 * W), lambda i: (i, 0)),
            pl.BlockSpec((R, 2 * W, 1), lambda i: (i, 0, 0)),
            pl.BlockSpec((R, 2 * W), lambda i: (i, 0)),
            pl.BlockSpec((64, POS_DIM), lambda i: (0, 0)),
        ],
        out_specs=[
            pl.BlockSpec((R, M, POS_DIM), lambda i: (i, 0, 0)),
            pl.BlockSpec((R, M, POS_DIM), lambda i: (i, 0, 0)),
        ],
        out_shape=out_shape,
    )(cat_flat, cat_col, cat_pos, emb_mat)

    return (src_pos.reshape(B, K, L, POS_DIM),
            tgt_pos.reshape(B, K, L, POS_DIM))
